# baseline (speedup 1.0000x reference)
import functools

import jax
import jax.numpy as jnp
from jax import lax
from jax.experimental import pallas as pl
from jax.experimental.pallas import tpu as pltpu
from jax.experimental.pallas import tpu_sc as plsc


def _proj_body(rel_ref, w_ref, t_ref):
    s = jnp.sum(rel_ref[...] * w_ref[...], axis=1)
    n = t_ref.shape[1]
    p = jnp.concatenate([s, jnp.zeros((n + 127 - s.shape[0],), jnp.float32)])
    for c in range(t_ref.shape[0]):
        t_ref[c, :] = lax.slice(p, (127 - c,), (127 - c + n,))


def _project_table(rel_pos, w_proj, L):
    return pl.pallas_call(
        _proj_body,
        out_shape=jax.ShapeDtypeStruct((128, 2 * L), jnp.float32),
    )(rel_pos, w_proj)


def _make_expand(L, NC, NS):
    NW = NC * NS
    n_groups = L // 8
    assert n_groups % NW == 0
    g_per_w = n_groups // NW
    n_h = n_groups // 16
    mesh = plsc.VectorSubcoreMesh(core_axis_name="c", subcore_axis_name="s")

    @functools.partial(
        pl.kernel,
        mesh=mesh,
        out_type=jax.ShapeDtypeStruct((L, L), jnp.float32),
        scratch_types=[pltpu.SemaphoreType.DMA],
    )
    def expand(t_hbm, out_hbm, sem):
        wid = lax.axis_index("s") * NC + lax.axis_index("c")
        copies = []
        for gg in range(g_per_w):
            G = wid * g_per_w + gg
            h = G // 16
            l = G % 16
            row = pl.multiple_of(8 * l, 8)
            col = pl.multiple_of(128 * (n_h - 1 - h), 128)
            row0 = pl.multiple_of(8 * G, 8)
            copies.append(
                pltpu.async_copy(
                    t_hbm.at[pl.ds(row, 8), pl.ds(col, L)],
                    out_hbm.at[pl.ds(row0, 8), :],
                    sem,
                )
            )
        for c in copies:
            c.wait()

    return expand


def kernel(rel_pos, W_proj, qlen, klen):
    L = (rel_pos.shape[0] + 1) // 2
    table = _project_table(rel_pos, W_proj, L)
    info = plsc.get_sparse_core_info()
    expand = _make_expand(L, info.num_cores, info.num_subcores)
    return expand(table)


# speedup vs baseline: 14.0226x; 14.0226x over previous
import functools

import jax
import jax.numpy as jnp
from jax import lax
from jax.experimental import pallas as pl
from jax.experimental.pallas import tpu as pltpu
from jax.experimental.pallas import tpu_sc as plsc


def _proj_body(rel_ref, w_ref, t_ref):
    s = jnp.sum(rel_ref[...] * w_ref[...], axis=1)
    n = t_ref.shape[1]
    p = jnp.concatenate([s, jnp.zeros((n + 127 - s.shape[0],), jnp.float32)])
    for c in range(t_ref.shape[0]):
        t_ref[c, :] = lax.slice(p, (127 - c,), (127 - c + n,))


def _project_table(rel_pos, w_proj, L):
    return pl.pallas_call(
        _proj_body,
        out_shape=jax.ShapeDtypeStruct((128, 2 * L), jnp.float32),
    )(rel_pos, w_proj)


def _make_expand(L, NC, NS):
    NW = NC * NS
    n_groups = L // 8
    assert n_groups % NW == 0
    g_per_w = n_groups // NW
    n_h = n_groups // 16
    mesh = plsc.VectorSubcoreMesh(core_axis_name="c", subcore_axis_name="s")

    @functools.partial(
        pl.kernel,
        mesh=mesh,
        out_type=jax.ShapeDtypeStruct((L, L), jnp.float32),
        scratch_types=[
            pltpu.VMEM_SHARED((128, 2 * L), jnp.float32),
            pltpu.SemaphoreType.DMA,
        ],
    )
    def expand(t_hbm, out_hbm, t_sp, sem):
        wid = lax.axis_index("s") * NC + lax.axis_index("c")
        sid = lax.axis_index("s")
        trow = pl.multiple_of(8 * sid, 8)
        pltpu.sync_copy(t_hbm.at[pl.ds(trow, 8), :], t_sp.at[pl.ds(trow, 8), :])
        plsc.subcore_barrier()
        copies = []
        for gg in range(g_per_w):
            G = wid * g_per_w + gg
            h = G // 16
            l = G % 16
            row = pl.multiple_of(8 * l, 8)
            col = pl.multiple_of(128 * (n_h - 1 - h), 128)
            row0 = pl.multiple_of(8 * G, 8)
            copies.append(
                pltpu.async_copy(
                    t_sp.at[pl.ds(row, 8), pl.ds(col, L)],
                    out_hbm.at[pl.ds(row0, 8), :],
                    sem,
                )
            )
        for c in copies:
            c.wait()

    return expand


def kernel(rel_pos, W_proj, qlen, klen):
    L = (rel_pos.shape[0] + 1) // 2
    table = _project_table(rel_pos, W_proj, L)
    info = plsc.get_sparse_core_info()
    expand = _make_expand(L, info.num_cores, info.num_subcores)
    return expand(table)


# speedup vs baseline: 14.7224x; 1.0499x over previous
import functools

import jax
import jax.numpy as jnp
from jax import lax
from jax.experimental import pallas as pl
from jax.experimental.pallas import tpu as pltpu
from jax.experimental.pallas import tpu_sc as plsc


def _proj_body(rel_ref, w_ref, t_ref):
    n = t_ref.shape[1]
    s_row = lax.dot_general(
        w_ref[...], rel_ref[...],
        (((1,), (1,)), ((), ())),
        preferred_element_type=jnp.float32,
    )
    p = jnp.concatenate([s_row, jnp.zeros((1, 128), jnp.float32)], axis=1)
    for c in range(t_ref.shape[0]):
        t_ref[c, :] = p[0, 127 - c : 127 - c + n]


def _project_table(rel_pos, w_proj, L):
    relp = jnp.pad(rel_pos, ((0, 1), (0, 0)))
    return pl.pallas_call(
        _proj_body,
        out_shape=jax.ShapeDtypeStruct((128, 2 * L), jnp.float32),
    )(relp, w_proj)


def _make_expand(L, NC, NS):
    NW = NC * NS
    n_groups = L // 8
    assert n_groups % NW == 0
    g_per_w = n_groups // NW
    n_h = n_groups // 16
    mesh = plsc.VectorSubcoreMesh(core_axis_name="c", subcore_axis_name="s")

    @functools.partial(
        pl.kernel,
        mesh=mesh,
        out_type=jax.ShapeDtypeStruct((L, L), jnp.float32),
        scratch_types=[
            pltpu.VMEM_SHARED((128, 2 * L), jnp.float32),
            pltpu.SemaphoreType.DMA,
        ],
    )
    def expand(t_hbm, out_hbm, t_sp, sem):
        wid = lax.axis_index("s") * NC + lax.axis_index("c")
        sid = lax.axis_index("s")
        trow = pl.multiple_of(8 * sid, 8)
        pltpu.sync_copy(t_hbm.at[pl.ds(trow, 8), :], t_sp.at[pl.ds(trow, 8), :])
        plsc.subcore_barrier()
        copies = []
        for gg in range(g_per_w):
            G = wid * g_per_w + gg
            h = G // 16
            l = G % 16
            row = pl.multiple_of(8 * l, 8)
            col = pl.multiple_of(128 * (n_h - 1 - h), 128)
            row0 = pl.multiple_of(8 * G, 8)
            copies.append(
                pltpu.async_copy(
                    t_sp.at[pl.ds(row, 8), pl.ds(col, L)],
                    out_hbm.at[pl.ds(row0, 8), :],
                    sem,
                )
            )
        for c in copies:
            c.wait()

    return expand


def kernel(rel_pos, W_proj, qlen, klen):
    L = (rel_pos.shape[0] + 1) // 2
    table = _project_table(rel_pos, W_proj, L)
    info = plsc.get_sparse_core_info()
    expand = _make_expand(L, info.num_cores, info.num_subcores)
    return expand(table)
